# Initial kernel scaffold; baseline (speedup 1.0000x reference)
import functools

import jax
import jax.numpy as jnp
from jax import lax
from jax.experimental import pallas as pl
from jax.experimental.pallas import tpu as pltpu
from jax.experimental.pallas import tpu_sc as plsc

VOCAB_ = 100000
EMB_ = 64
HID_ = 64
B_ = 1024
L_ = 50

_NC = 2
_NS = 16
_NW = _NC * _NS

_N_IDX = B_ * L_
_CHUNK = 100
_CHUNKS_TOTAL = _N_IDX // _CHUNK
_CHUNKS_PER_W = _CHUNKS_TOTAL // _NW


@functools.partial(
    pl.kernel,
    out_type=jax.ShapeDtypeStruct((_CHUNKS_TOTAL, _CHUNK, EMB_), jnp.float32),
    mesh=plsc.VectorSubcoreMesh(core_axis_name="c", subcore_axis_name="s"),
    scratch_types=[
        pltpu.VMEM((_CHUNKS_PER_W, _CHUNK), jnp.int32),
        pltpu.VMEM((_CHUNKS_PER_W, _CHUNK, EMB_), jnp.float32),
        pltpu.SemaphoreType.DMA,
    ],
)
def _sc_gather(idx_hbm, table_hbm, out_hbm, idx_v, rows_v, sem):
    wid = lax.axis_index("s") * _NC + lax.axis_index("c")
    base = wid * _CHUNKS_PER_W
    pltpu.sync_copy(idx_hbm.at[pl.ds(base, _CHUNKS_PER_W)], idx_v)
    for j in range(_CHUNKS_PER_W):
        pltpu.async_copy(table_hbm.at[idx_v.at[j]], rows_v.at[j], sem)
    pltpu.make_async_copy(out_hbm.at[pl.ds(base, _CHUNKS_PER_W)], rows_v, sem).wait()
    pltpu.sync_copy(rows_v, out_hbm.at[pl.ds(base, _CHUNKS_PER_W)])


def _rnn_step(x_ref, wih_ref, whh_ref, b_ref, out_ref, h_ref):
    t = pl.program_id(0)

    @pl.when(t == 0)
    def _():
        h_ref[...] = jnp.zeros_like(h_ref)

    x = x_ref[0]
    h = h_ref[...]
    a = jnp.dot(x, wih_ref[...], preferred_element_type=jnp.float32)
    s = jnp.dot(h, whh_ref[...], preferred_element_type=jnp.float32)
    hn = jnp.tanh(a + s + b_ref[...])
    h_ref[...] = hn
    out_ref[0] = hn


_rnn = pl.pallas_call(
    _rnn_step,
    grid=(L_,),
    in_specs=[
        pl.BlockSpec((1, B_, EMB_), lambda t: (t, 0, 0)),
        pl.BlockSpec((EMB_, HID_), lambda t: (0, 0)),
        pl.BlockSpec((HID_, HID_), lambda t: (0, 0)),
        pl.BlockSpec((1, HID_), lambda t: (0, 0)),
    ],
    out_specs=pl.BlockSpec((1, B_, HID_), lambda t: (t, 0, 0)),
    out_shape=jax.ShapeDtypeStruct((L_, B_, HID_), jnp.float32),
    scratch_shapes=[pltpu.VMEM((B_, HID_), jnp.float32)],
)


def kernel(sentences, emb_table, W_ih, W_hh, b_ih, b_hh):
    idx_tm = jnp.swapaxes(sentences, 0, 1).reshape(_CHUNKS_TOTAL, _CHUNK)
    idx_tm = idx_tm.astype(jnp.int32)
    x = _sc_gather(idx_tm, emb_table)
    x_tm = x.reshape(L_, B_, EMB_)
    bias = (b_ih + b_hh).reshape(1, HID_)
    ys = _rnn(x_tm, W_ih.T, W_hh.T, bias)
    final_output = jnp.swapaxes(ys, 0, 1)
    h = ys[L_ - 1][None, :, :]
    return final_output, h


# speedup vs baseline: 2.5254x; 2.5254x over previous
import functools

import jax
import jax.numpy as jnp
from jax import lax
from jax.experimental import pallas as pl
from jax.experimental.pallas import tpu as pltpu
from jax.experimental.pallas import tpu_sc as plsc

VOCAB_ = 100000
EMB_ = 64
HID_ = 64
B_ = 1024
L_ = 50

_NC = 2
_NS = 16
_NW = _NC * _NS

_N_IDX = B_ * L_
_CHUNK = 100
_CHUNKS_TOTAL = _N_IDX // _CHUNK
_CHUNKS_PER_W = _CHUNKS_TOTAL // _NW


def _sc_gather_body(idx_hbm, table_hbm, out_hbm, idx_v, rows_v, sem):
    wid = lax.axis_index("s") * _NC + lax.axis_index("c")
    base = wid * _CHUNKS_PER_W
    pltpu.sync_copy(idx_hbm.at[pl.ds(base, _CHUNKS_PER_W)], idx_v)
    for j in range(_CHUNKS_PER_W):
        pltpu.async_copy(table_hbm.at[idx_v.at[j]], rows_v.at[j], sem)
    pltpu.make_async_copy(out_hbm.at[pl.ds(base, _CHUNKS_PER_W)], rows_v, sem).wait()
    pltpu.sync_copy(rows_v, out_hbm.at[pl.ds(base, _CHUNKS_PER_W)])


@functools.lru_cache(maxsize=None)
def _sc_gather():
    return pl.kernel(
        _sc_gather_body,
        out_type=jax.ShapeDtypeStruct((_CHUNKS_TOTAL, _CHUNK, EMB_), jnp.float32),
        mesh=plsc.VectorSubcoreMesh(core_axis_name="c", subcore_axis_name="s"),
        scratch_types=[
            pltpu.VMEM((_CHUNKS_PER_W, _CHUNK), jnp.int32),
            pltpu.VMEM((_CHUNKS_PER_W, _CHUNK, EMB_), jnp.float32),
            pltpu.SemaphoreType.DMA,
        ],
        compiler_params=pltpu.CompilerParams(use_tc_tiling_on_sc=False),
    )


def _rnn_step(x_ref, wih_ref, whh_ref, b_ref, out_ref, h_ref):
    t = pl.program_id(0)

    @pl.when(t == 0)
    def _():
        h_ref[...] = jnp.zeros_like(h_ref)

    x = x_ref[0]
    h = h_ref[...]
    a = jnp.dot(x, wih_ref[...], preferred_element_type=jnp.float32)
    s = jnp.dot(h, whh_ref[...], preferred_element_type=jnp.float32)
    hn = jnp.tanh(a + s + b_ref[...])
    h_ref[...] = hn
    out_ref[0] = hn


_rnn = pl.pallas_call(
    _rnn_step,
    grid=(L_,),
    in_specs=[
        pl.BlockSpec((1, B_, EMB_), lambda t: (t, 0, 0)),
        pl.BlockSpec((EMB_, HID_), lambda t: (0, 0)),
        pl.BlockSpec((HID_, HID_), lambda t: (0, 0)),
        pl.BlockSpec((1, HID_), lambda t: (0, 0)),
    ],
    out_specs=pl.BlockSpec((1, B_, HID_), lambda t: (t, 0, 0)),
    out_shape=jax.ShapeDtypeStruct((L_, B_, HID_), jnp.float32),
    scratch_shapes=[pltpu.VMEM((B_, HID_), jnp.float32)],
)


def kernel(sentences, emb_table, W_ih, W_hh, b_ih, b_hh):
    idx_tm = jnp.swapaxes(sentences, 0, 1).reshape(_CHUNKS_TOTAL, _CHUNK)
    idx_tm = idx_tm.astype(jnp.int32)
    x = _sc_gather()(idx_tm, emb_table)
    x_tm = x.reshape(L_, B_, EMB_)
    bias = (b_ih + b_hh).reshape(1, HID_)
    ys = _rnn(x_tm, W_ih.T, W_hh.T, bias)
    final_output = jnp.swapaxes(ys, 0, 1)
    h = ys[L_ - 1][None, :, :]
    return final_output, h


# speedup vs baseline: 2.9050x; 1.1503x over previous
import functools

import jax
import jax.numpy as jnp
from jax import lax
from jax.experimental import pallas as pl
from jax.experimental.pallas import tpu as pltpu
from jax.experimental.pallas import tpu_sc as plsc

VOCAB_ = 100000
EMB_ = 64
HID_ = 64
B_ = 1024
L_ = 50

_NC = 2
_NS = 16
_NW = _NC * _NS

_N_IDX = B_ * L_
_CHUNK = 100
_CHUNKS_TOTAL = _N_IDX // _CHUNK
_CHUNKS_PER_W = _CHUNKS_TOTAL // _NW


def _sc_gather_body(idx_hbm, table_hbm, out_hbm, idx_v, rows_v, sem):
    wid = lax.axis_index("s") * _NC + lax.axis_index("c")
    base = wid * _CHUNKS_PER_W
    pltpu.sync_copy(idx_hbm.at[pl.ds(base, _CHUNKS_PER_W)], idx_v)
    for j in range(_CHUNKS_PER_W):
        pltpu.async_copy(table_hbm.at[idx_v.at[j]], rows_v.at[j], sem)
    pltpu.make_async_copy(out_hbm.at[pl.ds(base, _CHUNKS_PER_W)], rows_v, sem).wait()
    pltpu.sync_copy(rows_v, out_hbm.at[pl.ds(base, _CHUNKS_PER_W)])


@functools.lru_cache(maxsize=None)
def _sc_gather():
    return pl.kernel(
        _sc_gather_body,
        out_type=jax.ShapeDtypeStruct((_CHUNKS_TOTAL, _CHUNK, EMB_), jnp.float32),
        mesh=plsc.VectorSubcoreMesh(core_axis_name="c", subcore_axis_name="s"),
        scratch_types=[
            pltpu.VMEM((_CHUNKS_PER_W, _CHUNK), jnp.int32),
            pltpu.VMEM((_CHUNKS_PER_W, _CHUNK, EMB_), jnp.float32),
            pltpu.SemaphoreType.DMA,
        ],
        compiler_params=pltpu.CompilerParams(use_tc_tiling_on_sc=False),
    )


_BP = B_ // 2
_W2 = 2 * HID_


def _rnn_body(x_ref, wih_ref, whh_ref, b_ref, out_ref):
    x_all = x_ref[...].reshape(L_ * _BP, _W2)
    a = jnp.dot(x_all, wih_ref[...], preferred_element_type=jnp.float32)
    out_ref[...] = (a + b_ref[...]).reshape(L_, _BP, _W2)

    def step(t, h):
        hn = jnp.tanh(
            out_ref[t]
            + jnp.dot(h, whh_ref[...], preferred_element_type=jnp.float32)
        )
        out_ref[t] = hn
        return hn

    lax.fori_loop(0, L_, step, jnp.zeros((_BP, _W2), jnp.float32))


_rnn = pl.pallas_call(
    _rnn_body,
    out_shape=jax.ShapeDtypeStruct((L_, _BP, _W2), jnp.float32),
)


def _blockdiag2(w):
    z = jnp.zeros((HID_, HID_), w.dtype)
    return jnp.block([[w, z], [z, w]])


def kernel(sentences, emb_table, W_ih, W_hh, b_ih, b_hh):
    idx_tm = jnp.swapaxes(sentences, 0, 1).reshape(_CHUNKS_TOTAL, _CHUNK)
    idx_tm = idx_tm.astype(jnp.int32)
    x = _sc_gather()(idx_tm, emb_table)
    x2 = x.reshape(L_, _BP, _W2)
    bias1 = b_ih + b_hh
    bias2 = jnp.concatenate([bias1, bias1]).reshape(1, _W2)
    ys2 = _rnn(x2, _blockdiag2(W_ih.T), _blockdiag2(W_hh.T), bias2)
    ys = ys2.reshape(L_, B_, HID_)
    final_output = jnp.swapaxes(ys, 0, 1)
    h = ys[L_ - 1][None, :, :]
    return final_output, h


# speedup vs baseline: 2.9112x; 1.0021x over previous
import functools

import jax
import jax.numpy as jnp
import numpy as np
from jax import lax
from jax.experimental import pallas as pl
from jax.experimental.pallas import tpu as pltpu
from jax.experimental.pallas import tpu_sc as plsc

VOCAB_ = 100000
EMB_ = 64
HID_ = 64
B_ = 1024
L_ = 50

_NC = 2
_NS = 16
_NW = _NC * _NS

_N_IDX = B_ * L_
_CHUNK = 100
_CHUNKS_TOTAL = _N_IDX // _CHUNK
_CHUNKS_PER_W = _CHUNKS_TOTAL // _NW


def _sc_gather_body(idx_hbm, table_hbm, out_hbm, idx_v, rows_v, sem):
    wid = lax.axis_index("s") * _NC + lax.axis_index("c")
    base = wid * _CHUNKS_PER_W
    pltpu.sync_copy(idx_hbm.at[pl.ds(base, _CHUNKS_PER_W)], idx_v)
    for j in range(_CHUNKS_PER_W):
        pltpu.async_copy(table_hbm.at[idx_v.at[j]], rows_v.at[j], sem)
    pltpu.make_async_copy(out_hbm.at[pl.ds(base, _CHUNKS_PER_W)], rows_v, sem).wait()
    pltpu.sync_copy(rows_v, out_hbm.at[pl.ds(base, _CHUNKS_PER_W)])


@functools.lru_cache(maxsize=None)
def _sc_gather():
    return pl.kernel(
        _sc_gather_body,
        out_type=jax.ShapeDtypeStruct((_CHUNKS_TOTAL, _CHUNK, EMB_), jnp.float32),
        mesh=plsc.VectorSubcoreMesh(core_axis_name="c", subcore_axis_name="s"),
        scratch_types=[
            pltpu.VMEM((_CHUNKS_PER_W, _CHUNK), jnp.int32),
            pltpu.VMEM((_CHUNKS_PER_W, _CHUNK, EMB_), jnp.float32),
            pltpu.SemaphoreType.DMA,
        ],
        compiler_params=pltpu.CompilerParams(use_tc_tiling_on_sc=False),
    )


_BP = B_ // 2
_W2 = 2 * HID_


def _rnn_body(x_ref, wih_ref, whh_ref, b_ref, out_ref):
    x_all = x_ref[...].reshape(L_ * _BP, _W2)
    a = jnp.dot(x_all, wih_ref[...], preferred_element_type=jnp.float32)
    out_ref[...] = (a + b_ref[...]).reshape(L_, _BP, _W2)

    def step(t, h):
        hn = jnp.tanh(
            out_ref[t]
            + jnp.dot(h, whh_ref[...], preferred_element_type=jnp.float32)
        )
        out_ref[t] = hn
        return hn

    lax.fori_loop(0, L_, step, jnp.zeros((_BP, _W2), jnp.float32))


_rnn = pl.pallas_call(
    _rnn_body,
    out_shape=jax.ShapeDtypeStruct((L_, _BP, _W2), jnp.float32),
)


def _blockdiag2(w):
    z = jnp.zeros((HID_, HID_), w.dtype)
    return jnp.block([[w, z], [z, w]])


def kernel(sentences, emb_table, W_ih, W_hh, b_ih, b_hh):
    idx_tm = jax.lax.optimization_barrier(
        jnp.swapaxes(sentences.astype(jnp.int32), 0, 1))
    idx_tm = idx_tm.reshape(_CHUNKS_TOTAL, _CHUNK)
    x = _sc_gather()(idx_tm, emb_table)
    x2 = x.reshape(L_, _BP, _W2)
    bias1 = b_ih + b_hh
    bias2 = jnp.concatenate([bias1, bias1]).reshape(1, _W2)
    ys2 = _rnn(x2, _blockdiag2(W_ih.T), _blockdiag2(W_hh.T), bias2)
    ys = ys2.reshape(L_, B_, HID_)
    final_output = jnp.swapaxes(ys, 0, 1)
    h = ys[L_ - 1][None, :, :]
    return final_output, h
